# baseline (speedup 1.0000x reference)
import functools

import jax
import jax.numpy as jnp
from jax import lax
from jax.experimental import pallas as pl
from jax.experimental.pallas import tpu as pltpu
from jax.experimental.pallas import tpu_sc as plsc

_NEG_BIG = -1e30


def _sc_amax_body(a_ref, lab_ref, out_ref, acc_v, buf_v, lab_v, sem_a, sem_b,
                  *, num_protos, pix_per_w):
    c = lax.axis_index("c")
    s = lax.axis_index("s")
    wid = s * 2 + c
    b = wid // 2
    col = (wid % 2) * pix_per_w
    row0 = b * num_protos
    sems = (sem_a, sem_b)

    pltpu.sync_copy(a_ref.at[row0, pl.ds(col, pix_per_w)], acc_v)
    handles = [None, None]
    handles[0] = pltpu.async_copy(
        a_ref.at[row0 + 1, pl.ds(col, pix_per_w)], buf_v.at[0], sem_a)

    for p in range(1, num_protos):
        cur = (p - 1) % 2
        handles[cur].wait()
        if p + 1 < num_protos:
            nxt = p % 2
            handles[nxt] = pltpu.async_copy(
                a_ref.at[row0 + p + 1, pl.ds(col, pix_per_w)],
                buf_v.at[nxt], sems[nxt])

        def _maxbody(j, _, cur=cur):
            for u in range(8):
                off = j * 128 + u * 16
                acc_v[pl.ds(off, 16)] = jnp.maximum(
                    acc_v[pl.ds(off, 16)], buf_v[cur, pl.ds(off, 16)])
            return 0

        lax.fori_loop(0, pix_per_w // 128, _maxbody, 0)

    pltpu.sync_copy(lab_ref.at[pl.ds(wid * pix_per_w, pix_per_w)], lab_v)

    def _maskbody(j, _):
        for u in range(8):
            off = j * 128 + u * 16
            m = lab_v[pl.ds(off, 16)] == 1
            acc_v[pl.ds(off, 16)] = jnp.where(
                m, acc_v[pl.ds(off, 16)], _NEG_BIG)
        return 0

    lax.fori_loop(0, pix_per_w // 128, _maskbody, 0)
    pltpu.sync_copy(acc_v, out_ref.at[pl.ds(wid * pix_per_w, pix_per_w)])


def _sc_masked_amax(a_flat, lab_flat, num_protos, npix):
    pix_per_w = npix // 32
    fn = functools.partial(
        pl.kernel,
        mesh=plsc.VectorSubcoreMesh(core_axis_name="c", subcore_axis_name="s"),
        out_type=jax.ShapeDtypeStruct((npix,), jnp.float32),
        scratch_types=[
            pltpu.VMEM((pix_per_w,), jnp.float32),
            pltpu.VMEM((2, pix_per_w), jnp.float32),
            pltpu.VMEM((pix_per_w,), jnp.int32),
            pltpu.SemaphoreType.DMA,
            pltpu.SemaphoreType.DMA,
        ],
    )(functools.partial(_sc_amax_body, num_protos=num_protos,
                        pix_per_w=pix_per_w))
    return fn(a_flat, lab_flat)


def _tc_min_kernel(d_ref, l_ref, md_ref, cnt_ref):
    i = pl.program_id(0)
    p = d_ref.shape[1]

    @pl.when(i == 0)
    def _init():
        cnt_ref[...] = jnp.zeros_like(cnt_ref)

    d = d_ref[...]
    lab = l_ref[...]
    mask = lab == 1

    pmin = jnp.min(d, axis=1)
    md_ref[...] = jnp.where(mask, pmin, -1.0)

    iota_p = jax.lax.broadcasted_iota(jnp.int32, d.shape, 1)
    first_idx = jnp.min(jnp.where(d == pmin[:, None], iota_p, p), axis=1)
    onehot = (iota_p == first_idx[:, None]).astype(jnp.float32)
    contrib = jnp.where(mask[:, None], onehot, 0.0)
    cnt_ref[...] += jnp.sum(contrib, axis=(0, 2))


def _tc_min(distances, labels32):
    b, p, h, w = distances.shape
    nb = 2
    return pl.pallas_call(
        _tc_min_kernel,
        grid=(b // nb,),
        in_specs=[
            pl.BlockSpec((nb, p, h, w), lambda i: (i, 0, 0, 0)),
            pl.BlockSpec((nb, h, w), lambda i: (i, 0, 0)),
        ],
        out_specs=[
            pl.BlockSpec((nb, h, w), lambda i: (i, 0, 0)),
            pl.BlockSpec((p, w), lambda i: (0, 0)),
        ],
        out_shape=[
            jax.ShapeDtypeStruct((b, h, w), jnp.float32),
            jax.ShapeDtypeStruct((p, w), jnp.float32),
        ],
    )(distances, labels32)


def _tc_final_kernel(md_ref, ma_ref, cnt_ref, pv_ref, o_ref):
    p = cnt_ref.shape[0]
    md = md_ref[...]
    ma = ma_ref[...]
    counts = cnt_ref[...]

    n_f = jnp.sum(counts)
    n = n_f.astype(jnp.int32)
    safe_n = jnp.maximum(n_f, 1.0)
    base = jnp.sum(jnp.sum(jnp.maximum(md, 0.0), axis=0)) / safe_n

    k = jnp.maximum(1, (3 * n) // 10)
    k_f = k.astype(jnp.float32)

    def search_bit(j, prefix):
        cand = prefix | (jnp.int32(1) << (30 - j))
        th = jax.lax.bitcast_convert_type(cand, jnp.float32)
        ind = jnp.where(md >= th, 1.0, 0.0)
        cnt = jnp.sum(jnp.sum(ind, axis=0))
        return jnp.where(cnt >= k_f, cand, prefix)

    prefix = jax.lax.fori_loop(0, 23, search_bit, jnp.int32(0))
    t = jax.lax.bitcast_convert_type(prefix, jnp.float32)
    hard_sum = jnp.sum(jnp.sum(jnp.maximum(md - t, 0.0), axis=0)) + k_f * t
    hard_loss = (hard_sum / k_f) * 2.0

    usage = jnp.sum(counts, axis=1) / n_f
    entropy = -jnp.sum(usage * jnp.log(usage + 1e-8))
    max_entropy = jnp.log(jnp.float32(p))
    usage_div = (max_entropy - entropy) * 0.1

    m = jnp.max(ma)
    e = jnp.exp(ma - m)
    z = jnp.sum(jnp.sum(e, axis=0))
    wcl = jnp.sum(jnp.sum(md * e, axis=0)) / z

    cluster = base + hard_loss + usage_div + 0.5 * wcl
    cluster = jnp.where(n > 0, cluster, 0.0)

    pv = pv_ref[...]
    norms = jnp.maximum(
        jnp.sqrt(jnp.sum(pv * pv, axis=1, keepdims=True)), 1e-12)
    nv = pv / norms
    sim = jnp.dot(nv, nv.T, preferred_element_type=jnp.float32)
    rows = jax.lax.broadcasted_iota(jnp.int32, sim.shape, 0)
    cols = jax.lax.broadcasted_iota(jnp.int32, sim.shape, 1)
    offdiag = jnp.where(rows == cols, 0.0, jnp.abs(sim))
    div_loss = jnp.sum(offdiag) / jnp.float32(p * p)

    total = cluster + 0.01 * div_loss
    o_ref[...] = jnp.full((1, 1), total, dtype=jnp.float32)


def _tc_final(md, ma, counts, prototype_vectors):
    b, h, w = md.shape
    p, d = prototype_vectors.shape
    out = pl.pallas_call(
        _tc_final_kernel,
        in_specs=[
            pl.BlockSpec((b, h, w), lambda: (0, 0, 0)),
            pl.BlockSpec((b, h, w), lambda: (0, 0, 0)),
            pl.BlockSpec((p, w), lambda: (0, 0)),
            pl.BlockSpec((p, d), lambda: (0, 0)),
        ],
        out_specs=pl.BlockSpec((1, 1), lambda: (0, 0)),
        out_shape=jax.ShapeDtypeStruct((1, 1), jnp.float32),
    )(md, ma, counts, prototype_vectors)
    return out[0, 0]


def kernel(distances, activations, labels, prototype_vectors):
    b, p, h, w = distances.shape
    npix = b * h * w
    labels32 = labels.astype(jnp.int32)
    lab_flat = labels32.reshape(npix)
    a_flat = activations.reshape(b * p, h * w)

    ma_flat = _sc_masked_amax(a_flat, lab_flat, p, npix)
    md, counts = _tc_min(distances, labels32)
    ma = ma_flat.reshape(b, h, w)
    return _tc_final(md, ma, counts, prototype_vectors)


# speedup vs baseline: 2.1862x; 2.1862x over previous
import functools

import jax
import jax.numpy as jnp
from jax import lax
from jax.experimental import pallas as pl
from jax.experimental.pallas import tpu as pltpu
from jax.experimental.pallas import tpu_sc as plsc

_NEG_BIG = -1e30


_ROWS_PER_W = 64
_CHUNK_ROWS = 8


def _sc_amax_body(a_ref, lab_ref, out_ref, acc_v, buf_v, lab_v, sem_a, sem_b,
                  *, num_protos):
    c = lax.axis_index("c")
    s = lax.axis_index("s")
    wid = s * 2 + c
    b = wid // 2
    r0 = (wid % 2) * _ROWS_PER_W
    sems = (sem_a, sem_b)
    nchunk = _ROWS_PER_W // _CHUNK_ROWS

    pltpu.sync_copy(lab_ref.at[b, pl.ds(r0, _ROWS_PER_W)], lab_v)
    handles = [None, None]
    handles[0] = pltpu.async_copy(
        a_ref.at[b, :, pl.ds(r0, _CHUNK_ROWS)], buf_v.at[0], sem_a)

    for ch in range(nchunk):
        cur = ch % 2
        handles[cur].wait()
        if ch + 1 < nchunk:
            nxt = (ch + 1) % 2
            handles[nxt] = pltpu.async_copy(
                a_ref.at[b, :, pl.ds(r0 + (ch + 1) * _CHUNK_ROWS, _CHUNK_ROWS)],
                buf_v.at[nxt], sems[nxt])

        def _maxbody(row, _, cur=cur, ch=ch):
            for g in range(8):
                cs = pl.ds(g * 16, 16)
                r = buf_v[cur, 0, row, cs]
                for p in range(1, num_protos):
                    r = jnp.maximum(r, buf_v[cur, p, row, cs])
                acc_v[ch * _CHUNK_ROWS + row, cs] = r
            return 0

        lax.fori_loop(0, _CHUNK_ROWS, _maxbody, 0)

    def _maskbody(row, _):
        for g in range(8):
            cs = pl.ds(g * 16, 16)
            m = lab_v[row, cs] == 1
            acc_v[row, cs] = jnp.where(m, acc_v[row, cs], _NEG_BIG)
        return 0

    lax.fori_loop(0, _ROWS_PER_W, _maskbody, 0)
    pltpu.sync_copy(acc_v, out_ref.at[b, pl.ds(r0, _ROWS_PER_W)])


def _sc_masked_amax(activations, labels32):
    b, p, h, w = activations.shape
    fn = functools.partial(
        pl.kernel,
        mesh=plsc.VectorSubcoreMesh(core_axis_name="c", subcore_axis_name="s"),
        out_type=jax.ShapeDtypeStruct((b, h, w), jnp.float32),
        scratch_types=[
            pltpu.VMEM((_ROWS_PER_W, w), jnp.float32),
            pltpu.VMEM((2, p, _CHUNK_ROWS, w), jnp.float32),
            pltpu.VMEM((_ROWS_PER_W, w), jnp.int32),
            pltpu.SemaphoreType.DMA,
            pltpu.SemaphoreType.DMA,
        ],
        compiler_params=pltpu.CompilerParams(use_tc_tiling_on_sc=True),
    )(functools.partial(_sc_amax_body, num_protos=p))
    return fn(activations, labels32)


def _tc_min_kernel(d_ref, l_ref, md_ref, cnt_ref):
    i = pl.program_id(0)
    p = d_ref.shape[1]

    @pl.when(i == 0)
    def _init():
        cnt_ref[...] = jnp.zeros_like(cnt_ref)

    d = d_ref[...]
    lab = l_ref[...]
    mask = lab == 1

    pmin = jnp.min(d, axis=1)
    md_ref[...] = jnp.where(mask, pmin, -1.0)

    iota_p = jax.lax.broadcasted_iota(jnp.int32, d.shape, 1)
    first_idx = jnp.min(jnp.where(d == pmin[:, None], iota_p, p), axis=1)
    onehot = (iota_p == first_idx[:, None]).astype(jnp.float32)
    contrib = jnp.where(mask[:, None], onehot, 0.0)
    cnt_ref[...] += jnp.sum(contrib, axis=(0, 2))


def _tc_min(distances, labels32):
    b, p, h, w = distances.shape
    nb = 2
    return pl.pallas_call(
        _tc_min_kernel,
        grid=(b // nb,),
        in_specs=[
            pl.BlockSpec((nb, p, h, w), lambda i: (i, 0, 0, 0)),
            pl.BlockSpec((nb, h, w), lambda i: (i, 0, 0)),
        ],
        out_specs=[
            pl.BlockSpec((nb, h, w), lambda i: (i, 0, 0)),
            pl.BlockSpec((p, w), lambda i: (0, 0)),
        ],
        out_shape=[
            jax.ShapeDtypeStruct((b, h, w), jnp.float32),
            jax.ShapeDtypeStruct((p, w), jnp.float32),
        ],
    )(distances, labels32)


def _tc_final_kernel(md_ref, ma_ref, cnt_ref, pv_ref, o_ref):
    p = cnt_ref.shape[0]
    md = md_ref[...]
    ma = ma_ref[...]
    counts = cnt_ref[...]

    n_f = jnp.sum(counts)
    n = n_f.astype(jnp.int32)
    safe_n = jnp.maximum(n_f, 1.0)
    base = jnp.sum(jnp.sum(jnp.maximum(md, 0.0), axis=0)) / safe_n

    k = jnp.maximum(1, (3 * n) // 10)
    k_f = k.astype(jnp.float32)

    def search_bit(j, prefix):
        cand = prefix | (jnp.int32(1) << (30 - j))
        th = jax.lax.bitcast_convert_type(cand, jnp.float32)
        ind = jnp.where(md >= th, 1.0, 0.0)
        cnt = jnp.sum(jnp.sum(ind, axis=0))
        return jnp.where(cnt >= k_f, cand, prefix)

    prefix = jax.lax.fori_loop(0, 23, search_bit, jnp.int32(0))
    t = jax.lax.bitcast_convert_type(prefix, jnp.float32)
    hard_sum = jnp.sum(jnp.sum(jnp.maximum(md - t, 0.0), axis=0)) + k_f * t
    hard_loss = (hard_sum / k_f) * 2.0

    usage = jnp.sum(counts, axis=1) / n_f
    entropy = -jnp.sum(usage * jnp.log(usage + 1e-8))
    max_entropy = jnp.log(jnp.float32(p))
    usage_div = (max_entropy - entropy) * 0.1

    m = jnp.max(ma)
    e = jnp.exp(ma - m)
    z = jnp.sum(jnp.sum(e, axis=0))
    wcl = jnp.sum(jnp.sum(md * e, axis=0)) / z

    cluster = base + hard_loss + usage_div + 0.5 * wcl
    cluster = jnp.where(n > 0, cluster, 0.0)

    pv = pv_ref[...]
    norms = jnp.maximum(
        jnp.sqrt(jnp.sum(pv * pv, axis=1, keepdims=True)), 1e-12)
    nv = pv / norms
    sim = jnp.dot(nv, nv.T, preferred_element_type=jnp.float32)
    rows = jax.lax.broadcasted_iota(jnp.int32, sim.shape, 0)
    cols = jax.lax.broadcasted_iota(jnp.int32, sim.shape, 1)
    offdiag = jnp.where(rows == cols, 0.0, jnp.abs(sim))
    div_loss = jnp.sum(offdiag) / jnp.float32(p * p)

    total = cluster + 0.01 * div_loss
    o_ref[...] = jnp.full((1, 1), total, dtype=jnp.float32)


def _tc_final(md, ma, counts, prototype_vectors):
    b, h, w = md.shape
    p, d = prototype_vectors.shape
    out = pl.pallas_call(
        _tc_final_kernel,
        in_specs=[
            pl.BlockSpec((b, h, w), lambda: (0, 0, 0)),
            pl.BlockSpec((b, h, w), lambda: (0, 0, 0)),
            pl.BlockSpec((p, w), lambda: (0, 0)),
            pl.BlockSpec((p, d), lambda: (0, 0)),
        ],
        out_specs=pl.BlockSpec((1, 1), lambda: (0, 0)),
        out_shape=jax.ShapeDtypeStruct((1, 1), jnp.float32),
    )(md, ma, counts, prototype_vectors)
    return out[0, 0]


def kernel(distances, activations, labels, prototype_vectors):
    b, p, h, w = distances.shape
    labels32 = labels.astype(jnp.int32)

    ma = _sc_masked_amax(activations, labels32)
    md, counts = _tc_min(distances, labels32)
    return _tc_final(md, ma, counts, prototype_vectors)


# speedup vs baseline: 4.2447x; 1.9416x over previous
import functools

import jax
import jax.numpy as jnp
from jax.experimental import pallas as pl
from jax.experimental.pallas import tpu as pltpu

_NEG_BIG = -1e30


def _loss_kernel(d_ref, a_ref, l_ref, pv_ref, o_ref, md_ref, cnt_ref, acc_ref,
                 *, num_steps):
    i = pl.program_id(0)
    p = d_ref.shape[1]

    @pl.when(i == 0)
    def _init():
        cnt_ref[...] = jnp.zeros_like(cnt_ref)
        acc_ref[0] = _NEG_BIG
        acc_ref[1] = 0.0
        acc_ref[2] = 0.0
        acc_ref[3] = 0.0
        acc_ref[4] = 0.0

    nb = d_ref.shape[0]
    d = d_ref[...]
    a = a_ref[...]
    lab = l_ref[...]
    mask = lab == 1

    pmin = jnp.min(d, axis=1)
    amax = jnp.max(a, axis=1)
    md_step = jnp.where(mask, pmin, -1.0)
    ma_step = jnp.where(mask, amax, _NEG_BIG)

    iota_p = jax.lax.broadcasted_iota(jnp.int32, d.shape, 1)
    first_idx = jnp.min(jnp.where(d == pmin[:, None], iota_p, p), axis=1)
    onehot = (iota_p == first_idx[:, None]).astype(jnp.float32)
    contrib = jnp.where(mask[:, None], onehot, 0.0)
    cnt_ref[...] += jnp.sum(contrib, axis=(0, 2))

    md_ref[pl.ds(i * nb, nb)] = md_step

    m_old = acc_ref[0]
    m_new = jnp.maximum(m_old, jnp.max(ma_step))
    scale = jnp.exp(m_old - m_new)
    e_step = jnp.exp(ma_step - m_new)
    acc_ref[0] = m_new
    acc_ref[1] = acc_ref[1] * scale + jnp.sum(jnp.sum(e_step, axis=0))
    acc_ref[2] = acc_ref[2] * scale + jnp.sum(jnp.sum(md_step * e_step, axis=0))
    maskf = jnp.where(mask, 1.0, 0.0)
    acc_ref[3] += jnp.sum(jnp.sum(maskf, axis=0))
    acc_ref[4] += jnp.sum(jnp.sum(maskf * pmin, axis=0))

    @pl.when(i == num_steps - 1)
    def _finalize():
        md = md_ref[...]
        counts = cnt_ref[...]

        n_f = acc_ref[3]
        n = n_f.astype(jnp.int32)
        safe_n = jnp.maximum(n_f, 1.0)
        base = acc_ref[4] / safe_n

        k = jnp.maximum(1, (3 * n) // 10)
        k_f = k.astype(jnp.float32)

        def search_bit(j, prefix):
            cand = prefix | (jnp.int32(1) << (30 - j))
            th = jax.lax.bitcast_convert_type(cand, jnp.float32)
            ind = jnp.where(md >= th, 1.0, 0.0)
            cnt = jnp.sum(jnp.sum(ind, axis=0))
            return jnp.where(cnt >= k_f, cand, prefix)

        prefix = jax.lax.fori_loop(0, 23, search_bit, jnp.int32(0))
        t = jax.lax.bitcast_convert_type(prefix, jnp.float32)
        hard_sum = jnp.sum(jnp.sum(jnp.maximum(md - t, 0.0), axis=0)) + k_f * t
        hard_loss = (hard_sum / k_f) * 2.0

        usage = jnp.sum(counts, axis=1) / n_f
        entropy = -jnp.sum(usage * jnp.log(usage + 1e-8))
        max_entropy = jnp.log(jnp.float32(p))
        usage_div = (max_entropy - entropy) * 0.1

        wcl = acc_ref[2] / acc_ref[1]

        cluster = base + hard_loss + usage_div + 0.5 * wcl
        cluster = jnp.where(n > 0, cluster, 0.0)

        pv = pv_ref[...]
        norms = jnp.maximum(
            jnp.sqrt(jnp.sum(pv * pv, axis=1, keepdims=True)), 1e-12)
        nv = pv / norms
        sim = jnp.dot(nv, nv.T, preferred_element_type=jnp.float32)
        rows = jax.lax.broadcasted_iota(jnp.int32, sim.shape, 0)
        cols = jax.lax.broadcasted_iota(jnp.int32, sim.shape, 1)
        offdiag = jnp.where(rows == cols, 0.0, jnp.abs(sim))
        div_loss = jnp.sum(offdiag) / jnp.float32(p * p)

        total = cluster + 0.01 * div_loss
        o_ref[...] = jnp.full((1, 1), total, dtype=jnp.float32)


def kernel(distances, activations, labels, prototype_vectors):
    b, p, h, w = distances.shape
    d = prototype_vectors.shape[1]
    labels32 = labels.astype(jnp.int32)
    nb = 2
    out = pl.pallas_call(
        functools.partial(_loss_kernel, num_steps=b // nb),
        grid=(b // nb,),
        in_specs=[
            pl.BlockSpec((nb, p, h, w), lambda i: (i, 0, 0, 0)),
            pl.BlockSpec((nb, p, h, w), lambda i: (i, 0, 0, 0)),
            pl.BlockSpec((nb, h, w), lambda i: (i, 0, 0)),
            pl.BlockSpec((p, d), lambda i: (0, 0)),
        ],
        out_specs=pl.BlockSpec((1, 1), lambda i: (0, 0)),
        out_shape=jax.ShapeDtypeStruct((1, 1), jnp.float32),
        scratch_shapes=[
            pltpu.VMEM((b, h, w), jnp.float32),
            pltpu.VMEM((p, w), jnp.float32),
            pltpu.SMEM((8,), jnp.float32),
        ],
    )(distances, activations, labels32, prototype_vectors)
    return out[0, 0]


# speedup vs baseline: 4.3147x; 1.0165x over previous
import functools

import jax
import jax.numpy as jnp
from jax.experimental import pallas as pl
from jax.experimental.pallas import tpu as pltpu

_NEG_BIG = -1e30


def _loss_kernel(d_ref, a_ref, l_ref, pv_ref, o_ref, md_ref, cnt_ref, acc_ref,
                 *, num_steps):
    i = pl.program_id(0)
    p = d_ref.shape[1]

    @pl.when(i == 0)
    def _init():
        cnt_ref[...] = jnp.zeros_like(cnt_ref)
        acc_ref[0] = _NEG_BIG
        acc_ref[1] = 0.0
        acc_ref[2] = 0.0
        acc_ref[3] = 0.0
        acc_ref[4] = 0.0

    nb = d_ref.shape[0]
    d = d_ref[...]
    a = a_ref[...]
    lab = l_ref[...]
    mask = lab == 1

    pmin = jnp.min(d, axis=1)
    amax = jnp.max(a, axis=1)
    md_step = jnp.where(mask, pmin, -1.0)
    ma_step = jnp.where(mask, amax, _NEG_BIG)

    iota_p = jax.lax.broadcasted_iota(jnp.int32, d.shape, 1)
    first_idx = jnp.min(jnp.where(d == pmin[:, None], iota_p, p), axis=1)
    onehot = (iota_p == first_idx[:, None]).astype(jnp.float32)
    contrib = jnp.where(mask[:, None], onehot, 0.0)
    cnt_ref[...] += jnp.sum(contrib, axis=(0, 2))

    md_ref[pl.ds(i * nb, nb)] = md_step

    m_old = acc_ref[0]
    m_new = jnp.maximum(m_old, jnp.max(ma_step))
    scale = jnp.exp(m_old - m_new)
    e_step = jnp.exp(ma_step - m_new)
    acc_ref[0] = m_new
    acc_ref[1] = acc_ref[1] * scale + jnp.sum(jnp.sum(e_step, axis=0))
    acc_ref[2] = acc_ref[2] * scale + jnp.sum(jnp.sum(md_step * e_step, axis=0))
    maskf = jnp.where(mask, 1.0, 0.0)
    acc_ref[3] += jnp.sum(jnp.sum(maskf, axis=0))
    acc_ref[4] += jnp.sum(jnp.sum(maskf * pmin, axis=0))

    @pl.when(i == num_steps - 1)
    def _finalize():
        md = md_ref[...]
        counts = cnt_ref[...]

        n_f = acc_ref[3]
        n = n_f.astype(jnp.int32)
        safe_n = jnp.maximum(n_f, 1.0)
        base = acc_ref[4] / safe_n

        k = jnp.maximum(1, (3 * n) // 10)
        k_f = k.astype(jnp.float32)

        def search_bit(j, prefix):
            cand = prefix | (jnp.int32(1) << (30 - j))
            th = jax.lax.bitcast_convert_type(cand, jnp.float32)
            ind = jnp.where(md >= th, 1.0, 0.0)
            cnt = jnp.sum(jnp.sum(ind, axis=0))
            return jnp.where(cnt >= k_f, cand, prefix)

        prefix = jax.lax.fori_loop(0, 21, search_bit, jnp.int32(0))
        t = jax.lax.bitcast_convert_type(prefix, jnp.float32)
        hard_sum = jnp.sum(jnp.sum(jnp.maximum(md - t, 0.0), axis=0)) + k_f * t
        hard_loss = (hard_sum / k_f) * 2.0

        usage = jnp.sum(counts, axis=1) / n_f
        entropy = -jnp.sum(usage * jnp.log(usage + 1e-8))
        max_entropy = jnp.log(jnp.float32(p))
        usage_div = (max_entropy - entropy) * 0.1

        wcl = acc_ref[2] / acc_ref[1]

        cluster = base + hard_loss + usage_div + 0.5 * wcl
        cluster = jnp.where(n > 0, cluster, 0.0)

        pv = pv_ref[...]
        norms = jnp.maximum(
            jnp.sqrt(jnp.sum(pv * pv, axis=1, keepdims=True)), 1e-12)
        nv = pv / norms
        sim = jnp.dot(nv, nv.T, preferred_element_type=jnp.float32)
        rows = jax.lax.broadcasted_iota(jnp.int32, sim.shape, 0)
        cols = jax.lax.broadcasted_iota(jnp.int32, sim.shape, 1)
        offdiag = jnp.where(rows == cols, 0.0, jnp.abs(sim))
        div_loss = jnp.sum(offdiag) / jnp.float32(p * p)

        total = cluster + 0.01 * div_loss
        o_ref[...] = jnp.full((1, 1), total, dtype=jnp.float32)


def kernel(distances, activations, labels, prototype_vectors):
    b, p, h, w = distances.shape
    d = prototype_vectors.shape[1]
    labels32 = labels.astype(jnp.int32)
    nb = 2
    out = pl.pallas_call(
        functools.partial(_loss_kernel, num_steps=b // nb),
        grid=(b // nb,),
        in_specs=[
            pl.BlockSpec((nb, p, h, w), lambda i: (i, 0, 0, 0)),
            pl.BlockSpec((nb, p, h, w), lambda i: (i, 0, 0, 0)),
            pl.BlockSpec((nb, h, w), lambda i: (i, 0, 0)),
            pl.BlockSpec((p, d), lambda i: (0, 0)),
        ],
        out_specs=pl.BlockSpec((1, 1), lambda i: (0, 0)),
        out_shape=jax.ShapeDtypeStruct((1, 1), jnp.float32),
        scratch_shapes=[
            pltpu.VMEM((b, h, w), jnp.float32),
            pltpu.VMEM((p, w), jnp.float32),
            pltpu.SMEM((8,), jnp.float32),
        ],
    )(distances, activations, labels32, prototype_vectors)
    return out[0, 0]


# speedup vs baseline: 4.4178x; 1.0239x over previous
import functools

import jax
import jax.numpy as jnp
from jax.experimental import pallas as pl
from jax.experimental.pallas import tpu as pltpu

_NEG_BIG = -1e30


def _loss_kernel(d_ref, a_ref, l_ref, pv_ref, o_ref, md_ref, cnt_ref, acc_ref,
                 *, num_steps):
    i = pl.program_id(0)
    p = d_ref.shape[1]

    @pl.when(i == 0)
    def _init():
        cnt_ref[...] = jnp.zeros_like(cnt_ref)
        acc_ref[0] = _NEG_BIG
        acc_ref[1] = 0.0
        acc_ref[2] = 0.0
        acc_ref[3] = 0.0
        acc_ref[4] = 0.0

    nb = d_ref.shape[0]
    d = d_ref[...]
    a = a_ref[...]
    lab = l_ref[...]
    mask = lab == 1

    pmin = d[:, 0]
    first_idx = jnp.zeros(pmin.shape, jnp.int32)
    for q in range(1, p):
        dq = d[:, q]
        better = dq < pmin
        pmin = jnp.where(better, dq, pmin)
        first_idx = jnp.where(better, q, first_idx)

    amax = jnp.max(a, axis=1)
    md_step = jnp.where(mask, pmin, -1.0)
    ma_step = jnp.where(mask, amax, _NEG_BIG)

    iota_p = jax.lax.broadcasted_iota(jnp.int32, d.shape, 1)
    onehot = (iota_p == first_idx[:, None]).astype(jnp.float32)
    contrib = jnp.where(mask[:, None], onehot, 0.0)
    cnt_ref[...] += jnp.sum(contrib, axis=(0, 2))

    md_ref[pl.ds(i * nb, nb)] = md_step

    m_old = acc_ref[0]
    m_new = jnp.maximum(m_old, jnp.max(ma_step))
    scale = jnp.exp(m_old - m_new)
    e_step = jnp.exp(ma_step - m_new)
    acc_ref[0] = m_new
    acc_ref[1] = acc_ref[1] * scale + jnp.sum(jnp.sum(e_step, axis=0))
    acc_ref[2] = acc_ref[2] * scale + jnp.sum(jnp.sum(md_step * e_step, axis=0))
    maskf = jnp.where(mask, 1.0, 0.0)
    acc_ref[3] += jnp.sum(jnp.sum(maskf, axis=0))
    acc_ref[4] += jnp.sum(jnp.sum(maskf * pmin, axis=0))

    @pl.when(i == num_steps - 1)
    def _finalize():
        md = md_ref[...]
        counts = cnt_ref[...]

        n_f = acc_ref[3]
        n = n_f.astype(jnp.int32)
        safe_n = jnp.maximum(n_f, 1.0)
        base = acc_ref[4] / safe_n

        k = jnp.maximum(1, (3 * n) // 10)
        k_f = k.astype(jnp.float32)

        def search_bit(j, prefix):
            cand = prefix | (jnp.int32(1) << (30 - j))
            th = jax.lax.bitcast_convert_type(cand, jnp.float32)
            ind = jnp.where(md >= th, 1.0, 0.0)
            cnt = jnp.sum(jnp.sum(ind, axis=0))
            return jnp.where(cnt >= k_f, cand, prefix)

        prefix = jax.lax.fori_loop(0, 21, search_bit, jnp.int32(0))
        t = jax.lax.bitcast_convert_type(prefix, jnp.float32)
        hard_sum = jnp.sum(jnp.sum(jnp.maximum(md - t, 0.0), axis=0)) + k_f * t
        hard_loss = (hard_sum / k_f) * 2.0

        usage = jnp.sum(counts, axis=1) / n_f
        entropy = -jnp.sum(usage * jnp.log(usage + 1e-8))
        max_entropy = jnp.log(jnp.float32(p))
        usage_div = (max_entropy - entropy) * 0.1

        wcl = acc_ref[2] / acc_ref[1]

        cluster = base + hard_loss + usage_div + 0.5 * wcl
        cluster = jnp.where(n > 0, cluster, 0.0)

        pv = pv_ref[...]
        norms = jnp.maximum(
            jnp.sqrt(jnp.sum(pv * pv, axis=1, keepdims=True)), 1e-12)
        nv = pv / norms
        sim = jnp.dot(nv, nv.T, preferred_element_type=jnp.float32)
        rows = jax.lax.broadcasted_iota(jnp.int32, sim.shape, 0)
        cols = jax.lax.broadcasted_iota(jnp.int32, sim.shape, 1)
        offdiag = jnp.where(rows == cols, 0.0, jnp.abs(sim))
        div_loss = jnp.sum(offdiag) / jnp.float32(p * p)

        total = cluster + 0.01 * div_loss
        o_ref[...] = jnp.full((1, 1), total, dtype=jnp.float32)


def kernel(distances, activations, labels, prototype_vectors):
    b, p, h, w = distances.shape
    d = prototype_vectors.shape[1]
    labels32 = labels.astype(jnp.int32)
    nb = 2
    out = pl.pallas_call(
        functools.partial(_loss_kernel, num_steps=b // nb),
        grid=(b // nb,),
        in_specs=[
            pl.BlockSpec((nb, p, h, w), lambda i: (i, 0, 0, 0)),
            pl.BlockSpec((nb, p, h, w), lambda i: (i, 0, 0, 0)),
            pl.BlockSpec((nb, h, w), lambda i: (i, 0, 0)),
            pl.BlockSpec((p, d), lambda i: (0, 0)),
        ],
        out_specs=pl.BlockSpec((1, 1), lambda i: (0, 0)),
        out_shape=jax.ShapeDtypeStruct((1, 1), jnp.float32),
        scratch_shapes=[
            pltpu.VMEM((b, h, w), jnp.float32),
            pltpu.VMEM((p, w), jnp.float32),
            pltpu.SMEM((8,), jnp.float32),
        ],
    )(distances, activations, labels32, prototype_vectors)
    return out[0, 0]


# speedup vs baseline: 4.5918x; 1.0394x over previous
import functools

import jax
import jax.numpy as jnp
from jax.experimental import pallas as pl
from jax.experimental.pallas import tpu as pltpu

_NEG_BIG = -1e30


def _loss_kernel(d_ref, a_ref, l_ref, pv_ref, o_ref, md_ref, cnt_ref, acc_ref,
                 *, num_steps):
    i = pl.program_id(0)
    p = d_ref.shape[1]

    @pl.when(i == 0)
    def _init():
        cnt_ref[...] = jnp.zeros_like(cnt_ref)
        acc_ref[0] = _NEG_BIG
        acc_ref[1] = 0.0
        acc_ref[2] = 0.0
        acc_ref[4] = 0.0

    nb = d_ref.shape[0]
    d = d_ref[...]
    a = a_ref[...]
    lab = l_ref[...]
    mask = lab == 1

    pmin = d[:, 0]
    first_idx = jnp.zeros(pmin.shape, jnp.int32)
    for q in range(1, p):
        dq = d[:, q]
        better = dq < pmin
        pmin = jnp.where(better, dq, pmin)
        first_idx = jnp.where(better, q, first_idx)

    amax = jnp.max(a, axis=1)
    md_step = jnp.where(mask, pmin, -1.0)
    ma_step = jnp.where(mask, amax, _NEG_BIG)

    idx_m = jnp.where(mask, first_idx, p)
    iota_p = jax.lax.broadcasted_iota(jnp.int32, d.shape, 1)
    contrib = jnp.where(iota_p == idx_m[:, None], 1.0, 0.0)
    cnt_ref[...] += jnp.sum(contrib, axis=(0, 2))

    md_ref[pl.ds(i * nb, nb)] = md_step

    m_old = acc_ref[0]
    m_new = jnp.maximum(m_old, jnp.max(ma_step))
    scale = jnp.exp(m_old - m_new)
    e_step = jnp.exp(ma_step - m_new)
    acc_ref[0] = m_new
    acc_ref[1] = acc_ref[1] * scale + jnp.sum(jnp.sum(e_step, axis=0))
    acc_ref[2] = acc_ref[2] * scale + jnp.sum(jnp.sum(md_step * e_step, axis=0))
    acc_ref[4] += jnp.sum(jnp.sum(jnp.maximum(md_step, 0.0), axis=0))

    @pl.when(i == num_steps - 1)
    def _finalize():
        md = md_ref[...]
        counts = cnt_ref[...]

        n_f = jnp.sum(counts)
        n = n_f.astype(jnp.int32)
        safe_n = jnp.maximum(n_f, 1.0)
        base = acc_ref[4] / safe_n

        k = jnp.maximum(1, (3 * n) // 10)
        k_f = k.astype(jnp.float32)

        def search_bit(j, prefix):
            cand = prefix | (jnp.int32(1) << (30 - j))
            th = jax.lax.bitcast_convert_type(cand, jnp.float32)
            ind = jnp.where(md >= th, 1.0, 0.0)
            cnt = jnp.sum(jnp.sum(ind, axis=0))
            return jnp.where(cnt >= k_f, cand, prefix)

        prefix = jax.lax.fori_loop(0, 21, search_bit, jnp.int32(0))
        t = jax.lax.bitcast_convert_type(prefix, jnp.float32)
        hard_sum = jnp.sum(jnp.sum(jnp.maximum(md - t, 0.0), axis=0)) + k_f * t
        hard_loss = (hard_sum / k_f) * 2.0

        usage = jnp.sum(counts, axis=1) / n_f
        entropy = -jnp.sum(usage * jnp.log(usage + 1e-8))
        max_entropy = jnp.log(jnp.float32(p))
        usage_div = (max_entropy - entropy) * 0.1

        wcl = acc_ref[2] / acc_ref[1]

        cluster = base + hard_loss + usage_div + 0.5 * wcl
        cluster = jnp.where(n > 0, cluster, 0.0)

        pv = pv_ref[...]
        norms = jnp.maximum(
            jnp.sqrt(jnp.sum(pv * pv, axis=1, keepdims=True)), 1e-12)
        nv = pv / norms
        sim = jnp.dot(nv, nv.T, preferred_element_type=jnp.float32)
        rows = jax.lax.broadcasted_iota(jnp.int32, sim.shape, 0)
        cols = jax.lax.broadcasted_iota(jnp.int32, sim.shape, 1)
        offdiag = jnp.where(rows == cols, 0.0, jnp.abs(sim))
        div_loss = jnp.sum(offdiag) / jnp.float32(p * p)

        total = cluster + 0.01 * div_loss
        o_ref[...] = jnp.full((1, 1), total, dtype=jnp.float32)


def kernel(distances, activations, labels, prototype_vectors):
    b, p, h, w = distances.shape
    d = prototype_vectors.shape[1]
    labels32 = labels.astype(jnp.int32)
    nb = 2
    out = pl.pallas_call(
        functools.partial(_loss_kernel, num_steps=b // nb),
        grid=(b // nb,),
        in_specs=[
            pl.BlockSpec((nb, p, h, w), lambda i: (i, 0, 0, 0)),
            pl.BlockSpec((nb, p, h, w), lambda i: (i, 0, 0, 0)),
            pl.BlockSpec((nb, h, w), lambda i: (i, 0, 0)),
            pl.BlockSpec((p, d), lambda i: (0, 0)),
        ],
        out_specs=pl.BlockSpec((1, 1), lambda i: (0, 0)),
        out_shape=jax.ShapeDtypeStruct((1, 1), jnp.float32),
        scratch_shapes=[
            pltpu.VMEM((b, h, w), jnp.float32),
            pltpu.VMEM((p, w), jnp.float32),
            pltpu.SMEM((8,), jnp.float32),
        ],
    )(distances, activations, labels32, prototype_vectors)
    return out[0, 0]


# speedup vs baseline: 4.6550x; 1.0137x over previous
import functools

import jax
import jax.numpy as jnp
from jax.experimental import pallas as pl
from jax.experimental.pallas import tpu as pltpu

_NEG_BIG = -1e30


def _loss_kernel(d_ref, a_ref, l_ref, pv_ref, o_ref, md_ref, cnt_ref, acc_ref,
                 *, num_steps):
    i = pl.program_id(0)
    p = d_ref.shape[1]

    @pl.when(i == 0)
    def _init():
        cnt_ref[...] = jnp.zeros_like(cnt_ref)
        acc_ref[0] = _NEG_BIG
        acc_ref[1] = 0.0
        acc_ref[2] = 0.0
        acc_ref[4] = 0.0

    nb = d_ref.shape[0]
    d = d_ref[...]
    a = a_ref[...]
    lab = l_ref[...]
    mask = lab == 1

    hh = d.shape[2] // 2
    pmins, idxs = [], []
    for bi in range(nb):
        for hi in range(2):
            hs = slice(hi * hh, (hi + 1) * hh)
            pm = d[bi, 0, hs]
            fi = jnp.zeros(pm.shape, jnp.int32)
            for q in range(1, p):
                dq = d[bi, q, hs]
                better = dq < pm
                pm = jnp.where(better, dq, pm)
                fi = jnp.where(better, q, fi)
            pmins.append(pm)
            idxs.append(fi)
    pmin = jnp.concatenate(
        [jnp.concatenate(pmins[2 * bi:2 * bi + 2], axis=0)[None]
         for bi in range(nb)], axis=0)
    first_idx = jnp.concatenate(
        [jnp.concatenate(idxs[2 * bi:2 * bi + 2], axis=0)[None]
         for bi in range(nb)], axis=0)

    amax = jnp.max(a, axis=1)
    md_step = jnp.where(mask, pmin, -1.0)
    ma_step = jnp.where(mask, amax, _NEG_BIG)

    idx_m = jnp.where(mask, first_idx, p)
    iota_p = jax.lax.broadcasted_iota(jnp.int32, d.shape, 1)
    contrib = jnp.where(iota_p == idx_m[:, None], 1.0, 0.0)
    cnt_ref[...] += jnp.sum(contrib, axis=(0, 2))

    md_ref[pl.ds(i * nb, nb)] = md_step

    m_old = acc_ref[0]
    m_new = jnp.maximum(m_old, jnp.max(ma_step))
    scale = jnp.exp(m_old - m_new)
    e_step = jnp.exp(ma_step - m_new)
    acc_ref[0] = m_new
    acc_ref[1] = acc_ref[1] * scale + jnp.sum(jnp.sum(e_step, axis=0))
    acc_ref[2] = acc_ref[2] * scale + jnp.sum(jnp.sum(md_step * e_step, axis=0))
    acc_ref[4] += jnp.sum(jnp.sum(jnp.maximum(md_step, 0.0), axis=0))

    @pl.when(i == num_steps - 1)
    def _finalize():
        md = md_ref[...]
        counts = cnt_ref[...]

        n_f = jnp.sum(counts)
        n = n_f.astype(jnp.int32)
        safe_n = jnp.maximum(n_f, 1.0)
        base = acc_ref[4] / safe_n

        k = jnp.maximum(1, (3 * n) // 10)
        k_f = k.astype(jnp.float32)

        def search_bit(j, prefix):
            cand = prefix | (jnp.int32(1) << (30 - j))
            th = jax.lax.bitcast_convert_type(cand, jnp.float32)
            ind = jnp.where(md >= th, 1.0, 0.0)
            cnt = jnp.sum(jnp.sum(ind, axis=0))
            return jnp.where(cnt >= k_f, cand, prefix)

        prefix = jax.lax.fori_loop(0, 21, search_bit, jnp.int32(0))
        t = jax.lax.bitcast_convert_type(prefix, jnp.float32)
        hard_sum = jnp.sum(jnp.sum(jnp.maximum(md - t, 0.0), axis=0)) + k_f * t
        hard_loss = (hard_sum / k_f) * 2.0

        usage = jnp.sum(counts, axis=1) / n_f
        entropy = -jnp.sum(usage * jnp.log(usage + 1e-8))
        max_entropy = jnp.log(jnp.float32(p))
        usage_div = (max_entropy - entropy) * 0.1

        wcl = acc_ref[2] / acc_ref[1]

        cluster = base + hard_loss + usage_div + 0.5 * wcl
        cluster = jnp.where(n > 0, cluster, 0.0)

        pv = pv_ref[...]
        norms = jnp.maximum(
            jnp.sqrt(jnp.sum(pv * pv, axis=1, keepdims=True)), 1e-12)
        nv = pv / norms
        sim = jnp.dot(nv, nv.T, preferred_element_type=jnp.float32)
        rows = jax.lax.broadcasted_iota(jnp.int32, sim.shape, 0)
        cols = jax.lax.broadcasted_iota(jnp.int32, sim.shape, 1)
        offdiag = jnp.where(rows == cols, 0.0, jnp.abs(sim))
        div_loss = jnp.sum(offdiag) / jnp.float32(p * p)

        total = cluster + 0.01 * div_loss
        o_ref[...] = jnp.full((1, 1), total, dtype=jnp.float32)


def kernel(distances, activations, labels, prototype_vectors):
    b, p, h, w = distances.shape
    d = prototype_vectors.shape[1]
    labels32 = labels.astype(jnp.int32)
    nb = 2
    out = pl.pallas_call(
        functools.partial(_loss_kernel, num_steps=b // nb),
        grid=(b // nb,),
        in_specs=[
            pl.BlockSpec((nb, p, h, w), lambda i: (i, 0, 0, 0)),
            pl.BlockSpec((nb, p, h, w), lambda i: (i, 0, 0, 0)),
            pl.BlockSpec((nb, h, w), lambda i: (i, 0, 0)),
            pl.BlockSpec((p, d), lambda i: (0, 0)),
        ],
        out_specs=pl.BlockSpec((1, 1), lambda i: (0, 0)),
        out_shape=jax.ShapeDtypeStruct((1, 1), jnp.float32),
        scratch_shapes=[
            pltpu.VMEM((b, h, w), jnp.float32),
            pltpu.VMEM((p, w), jnp.float32),
            pltpu.SMEM((8,), jnp.float32),
        ],
    )(distances, activations, labels32, prototype_vectors)
    return out[0, 0]
